# baseline (speedup 1.0000x reference)
import functools

import jax
import jax.numpy as jnp
from jax import lax
from jax.experimental import pallas as pl
from jax.experimental.pallas import tpu as pltpu
from jax.experimental.pallas import tpu_sc as plsc

D_MODEL = 768
D_FF = 1024
N_A = 2
N_B = 4
N_E = N_A * N_B
SEQ = 2048
BLK = 384
MAXB = 13
NSLOT = MAXB * BLK


def _gating_body(x_ref, gaw_ref, gab_ref, gbw_ref, gbb_ref,
                 slot_ref, be_ref, nb_ref, aux_ref):
    xv = x_ref[...]
    dn = (((1,), (1,)), ((), ()))
    la = lax.dot_general(xv, gaw_ref[...], dn,
                         preferred_element_type=jnp.float32) + gab_ref[...]
    lb = lax.dot_general(xv, gbw_ref[...], dn,
                         preferred_element_type=jnp.float32) + gbb_ref[...]

    pa = jax.nn.softmax(la, axis=-1)
    pb = jax.nn.softmax(lb, axis=-1)

    def first_argmax(p, n):
        m = jnp.max(p, axis=-1, keepdims=True)
        io = lax.broadcasted_iota(jnp.int32, p.shape, 1).astype(jnp.float32)
        return jnp.min(jnp.where(p == m, io, float(n)), axis=-1, keepdims=True)

    ia = first_argmax(pa, N_A)
    ib = first_argmax(pb, N_B)
    e_col = ia * N_B + ib

    ioe = lax.broadcasted_iota(jnp.int32, (SEQ, N_E), 1).astype(jnp.float32)
    oh = (ioe == e_col).astype(jnp.float32)

    ioa = lax.broadcasted_iota(jnp.int32, (SEQ, N_A), 1).astype(jnp.float32)
    iob = lax.broadcasted_iota(jnp.int32, (SEQ, N_B), 1).astype(jnp.float32)
    oha = (ioa == ia).astype(jnp.float32)
    ohb = (iob == ib).astype(jnp.float32)
    aux_a = N_A * jnp.sum(jnp.mean(pa, axis=0, keepdims=True)
                          * jnp.mean(oha, axis=0, keepdims=True))
    aux_b = N_B * jnp.sum(jnp.mean(pb, axis=0, keepdims=True)
                          * jnp.mean(ohb, axis=0, keepdims=True))
    aux_ref[...] = jnp.full((1, 1), 0.0) + aux_a + aux_b

    counts = jnp.sum(oh, axis=0, keepdims=True)
    counts8 = counts.reshape(N_E, 1)
    nblk8 = jnp.ceil(counts8 / float(BLK))
    r8 = lax.broadcasted_iota(jnp.int32, (N_E, N_E), 0).astype(jnp.float32)
    c8 = lax.broadcasted_iota(jnp.int32, (N_E, N_E), 1).astype(jnp.float32)
    lower_incl = (c8 <= r8).astype(jnp.float32)
    ends8 = jnp.dot(lower_incl, nblk8,
                    preferred_element_type=jnp.float32)
    off_rows8 = (ends8 - nblk8) * float(BLK)

    rc = 128
    rr = lax.broadcasted_iota(jnp.int32, (rc, rc), 0).astype(jnp.float32)
    cc = lax.broadcasted_iota(jnp.int32, (rc, rc), 1).astype(jnp.float32)
    tri = (cc < rr).astype(jnp.float32)
    base = jnp.zeros((1, N_E), jnp.float32)
    ranks = []
    for i in range(SEQ // rc):
        blk = lax.slice(oh, (i * rc, 0), ((i + 1) * rc, N_E))
        ranks.append(jnp.dot(tri, blk, preferred_element_type=jnp.float32)
                     + base)
        base = base + jnp.sum(blk, axis=0, keepdims=True)
    rank = jnp.concatenate(ranks, axis=0)

    off_tok = jnp.dot(oh, off_rows8,
                      preferred_element_type=jnp.float32)
    rank_tok = jnp.sum(oh * rank, axis=1, keepdims=True)
    slot = off_tok + rank_tok
    slot_ref[...] = slot.astype(jnp.int32).reshape(SEQ // 128, 128)

    iom = lax.broadcasted_iota(jnp.int32, (N_E, MAXB), 1).astype(jnp.float32)
    ended = (jnp.broadcast_to(ends8, (N_E, MAXB)) <= iom).astype(jnp.float32)
    be_raw = jnp.sum(ended, axis=0, keepdims=True)
    nb = lax.slice(ends8, (N_E - 1, 0), (N_E, 1))
    active = lax.broadcasted_iota(
        jnp.int32, (1, MAXB), 1).astype(jnp.float32) < nb
    be_last = jnp.max(jnp.where(active, be_raw, 0.0))
    be = jnp.minimum(be_raw, be_last)
    be_ref[...] = be.astype(jnp.int32)
    nb_ref[...] = nb.astype(jnp.int32)


def _gating_call(xf, gaw, gab, gbw, gbb):
    return pl.pallas_call(
        _gating_body,
        out_shape=[
            jax.ShapeDtypeStruct((SEQ // 128, 128), jnp.int32),
            jax.ShapeDtypeStruct((1, MAXB), jnp.int32),
            jax.ShapeDtypeStruct((1, 1), jnp.int32),
            jax.ShapeDtypeStruct((1, 1), jnp.float32),
        ],
    )(xf, gaw, gab, gbw, gbb)


_NC, _NS = 2, 16
_NW = _NC * _NS
_BPW = SEQ // _NW


@functools.cache
def _sc_kernels():
    mesh = plsc.VectorSubcoreMesh(core_axis_name="c", subcore_axis_name="s")

    @functools.partial(
        pl.kernel,
        mesh=mesh,
        out_type=jax.ShapeDtypeStruct((NSLOT, D_MODEL), jnp.float32),
        scratch_types=[
            pltpu.VMEM((_BPW,), jnp.int32),
            pltpu.VMEM((_BPW, D_MODEL), jnp.float32),
            pltpu.SemaphoreType.DMA,
        ],
    )
    def dispatch_scatter(x_hbm, slot_hbm, xs_hbm, idx_v, rows_v, sem):
        wid = lax.axis_index("s") * _NC + lax.axis_index("c")
        base = wid * _BPW
        pltpu.sync_copy(slot_hbm.at[pl.ds(base, _BPW)], idx_v)
        pltpu.sync_copy(x_hbm.at[pl.ds(base, _BPW)], rows_v)
        pltpu.async_copy(rows_v, xs_hbm.at[idx_v], sem).wait()

    @functools.partial(
        pl.kernel,
        mesh=mesh,
        out_type=jax.ShapeDtypeStruct((SEQ, D_MODEL), jnp.float32),
        scratch_types=[
            pltpu.VMEM((_BPW,), jnp.int32),
            pltpu.VMEM((_BPW, D_MODEL), jnp.float32),
            pltpu.SemaphoreType.DMA,
        ],
    )
    def combine_gather(ys_hbm, slot_hbm, out_hbm, idx_v, rows_v, sem):
        wid = lax.axis_index("s") * _NC + lax.axis_index("c")
        base = wid * _BPW
        pltpu.sync_copy(slot_hbm.at[pl.ds(base, _BPW)], idx_v)
        pltpu.async_copy(ys_hbm.at[idx_v], rows_v, sem).wait()
        pltpu.sync_copy(rows_v, out_hbm.at[pl.ds(base, _BPW)])

    return dispatch_scatter, combine_gather


def _ffn_body(be_ref, nb_ref, x_ref, w1a_ref, w1g_ref, b1_ref,
              w2a_ref, w2b_ref, b2_ref, o_ref):
    i = pl.program_id(0)

    @pl.when(i < nb_ref[0])
    def _():
        xb = x_ref[...]
        b1 = b1_ref[0]
        a = jnp.dot(xb, w1a_ref[0, :, 0, 0, :],
                    preferred_element_type=jnp.float32) + b1[:, :D_FF]
        g = jnp.dot(xb, w1g_ref[0, :, 0, 0, :],
                    preferred_element_type=jnp.float32) + b1[:, D_FF:]
        act = a * (g * lax.logistic(g))
        y = (jnp.dot(act[:, :D_FF // 2], w2a_ref[0, 0],
                     preferred_element_type=jnp.float32)
             + jnp.dot(act[:, D_FF // 2:], w2b_ref[0, 0],
                       preferred_element_type=jnp.float32))
        o_ref[...] = y + b2_ref[0]


def _ffn_call(be, nb, xs, w1, b1, w2, b2):
    grid_spec = pltpu.PrefetchScalarGridSpec(
        num_scalar_prefetch=2,
        grid=(MAXB,),
        in_specs=[
            pl.BlockSpec((BLK, D_MODEL),
                         lambda i, be, nb: (jnp.minimum(i, nb[0] - 1), 0)),
            pl.BlockSpec((1, D_MODEL, 1, 1, D_FF),
                         lambda i, be, nb: (be[i], 0, 0, 0, 0)),
            pl.BlockSpec((1, D_MODEL, 1, 1, D_FF),
                         lambda i, be, nb: (be[i], 0, 1, 0, 0)),
            pl.BlockSpec((1, 1, 2 * D_FF), lambda i, be, nb: (be[i], 0, 0)),
            pl.BlockSpec((1, 1, D_FF // 2, D_MODEL),
                         lambda i, be, nb: (be[i], 0, 0, 0)),
            pl.BlockSpec((1, 1, D_FF // 2, D_MODEL),
                         lambda i, be, nb: (be[i], 1, 0, 0)),
            pl.BlockSpec((1, 1, D_MODEL), lambda i, be, nb: (be[i], 0, 0)),
        ],
        out_specs=pl.BlockSpec((BLK, D_MODEL),
                               lambda i, be, nb: (jnp.minimum(i, nb[0] - 1), 0)),
    )
    return pl.pallas_call(
        _ffn_body,
        grid_spec=grid_spec,
        out_shape=jax.ShapeDtypeStruct((NSLOT, D_MODEL), jnp.float32),
    )(be, nb, xs, w1, w1, b1, w2, w2, b2)


def kernel(x, gA_w, gA_b, gB_w, gB_b, w1_w, w1_b, w2_w, w2_b):
    b, s, d = x.shape
    xf = x.reshape(s, d)

    slot2d, be2d, nb2d, aux2d = _gating_call(
        xf, gA_w.T, gA_b.reshape(1, N_A), gB_w.T, gB_b.reshape(1, N_B))
    slot = slot2d.reshape(s)
    be = be2d.reshape(MAXB)
    nb = nb2d.reshape(1)

    dispatch_scatter, combine_gather = _sc_kernels()
    xs = dispatch_scatter(xf, slot)

    w1 = w1_w.reshape(N_E, d, 2, 1, D_FF)
    b1 = w1_b.reshape(N_E, 1, 2 * D_FF)
    w2 = w2_w.reshape(N_E, 2, D_FF // 2, d)
    b2 = w2_b.reshape(N_E, 1, d)
    ys = _ffn_call(be, nb, xs, w1, b1, w2, b2)

    outf = combine_gather(ys, slot)
    return outf.reshape(b, s, d), aux2d.reshape(())


# speedup vs baseline: 3.2475x; 3.2475x over previous
import functools

import jax
import jax.numpy as jnp
from jax import lax
from jax.experimental import pallas as pl
from jax.experimental.pallas import tpu as pltpu
from jax.experimental.pallas import tpu_sc as plsc

D_MODEL = 768
D_FF = 1024
N_A = 2
N_B = 4
N_E = N_A * N_B
SEQ = 2048
BLK = 384
MAXB = 13
NSLOT = MAXB * BLK


def _gating_body(x_ref, gaw_ref, gab_ref, gbw_ref, gbb_ref,
                 slot_ref, be_ref, nb_ref, aux_ref):
    xv = x_ref[...]
    dn = (((1,), (1,)), ((), ()))
    la = lax.dot_general(xv, gaw_ref[...], dn,
                         preferred_element_type=jnp.float32) + gab_ref[...]
    lb = lax.dot_general(xv, gbw_ref[...], dn,
                         preferred_element_type=jnp.float32) + gbb_ref[...]

    pa = jax.nn.softmax(la, axis=-1)
    pb = jax.nn.softmax(lb, axis=-1)

    def first_argmax(p, n):
        m = jnp.max(p, axis=-1, keepdims=True)
        io = lax.broadcasted_iota(jnp.int32, p.shape, 1).astype(jnp.float32)
        return jnp.min(jnp.where(p == m, io, float(n)), axis=-1, keepdims=True)

    ia = first_argmax(pa, N_A)
    ib = first_argmax(pb, N_B)
    e_col = ia * N_B + ib

    ioe = lax.broadcasted_iota(jnp.int32, (SEQ, N_E), 1).astype(jnp.float32)
    oh = (ioe == e_col).astype(jnp.float32)

    ioa = lax.broadcasted_iota(jnp.int32, (SEQ, N_A), 1).astype(jnp.float32)
    iob = lax.broadcasted_iota(jnp.int32, (SEQ, N_B), 1).astype(jnp.float32)
    oha = (ioa == ia).astype(jnp.float32)
    ohb = (iob == ib).astype(jnp.float32)
    aux_a = N_A * jnp.sum(jnp.mean(pa, axis=0, keepdims=True)
                          * jnp.mean(oha, axis=0, keepdims=True))
    aux_b = N_B * jnp.sum(jnp.mean(pb, axis=0, keepdims=True)
                          * jnp.mean(ohb, axis=0, keepdims=True))
    aux_ref[...] = jnp.full((1, 1), 0.0) + aux_a + aux_b

    counts = jnp.sum(oh, axis=0, keepdims=True)
    counts8 = counts.reshape(N_E, 1)
    nblk8 = jnp.ceil(counts8 / float(BLK))
    r8 = lax.broadcasted_iota(jnp.int32, (N_E, N_E), 0).astype(jnp.float32)
    c8 = lax.broadcasted_iota(jnp.int32, (N_E, N_E), 1).astype(jnp.float32)
    lower_incl = (c8 <= r8).astype(jnp.float32)
    ends8 = jnp.dot(lower_incl, nblk8,
                    preferred_element_type=jnp.float32)
    off_rows8 = (ends8 - nblk8) * float(BLK)

    rc = 128
    rr = lax.broadcasted_iota(jnp.int32, (rc, rc), 0).astype(jnp.float32)
    cc = lax.broadcasted_iota(jnp.int32, (rc, rc), 1).astype(jnp.float32)
    tri = (cc < rr).astype(jnp.float32)
    base = jnp.zeros((1, N_E), jnp.float32)
    ranks = []
    for i in range(SEQ // rc):
        blk = lax.slice(oh, (i * rc, 0), ((i + 1) * rc, N_E))
        ranks.append(jnp.dot(tri, blk, preferred_element_type=jnp.float32)
                     + base)
        base = base + jnp.sum(blk, axis=0, keepdims=True)
    rank = jnp.concatenate(ranks, axis=0)

    off_tok = jnp.dot(oh, off_rows8,
                      preferred_element_type=jnp.float32)
    rank_tok = jnp.sum(oh * rank, axis=1, keepdims=True)
    slot = off_tok + rank_tok
    slot_ref[...] = slot.astype(jnp.int32).reshape(SEQ // 128, 128)

    iom = lax.broadcasted_iota(jnp.int32, (N_E, MAXB), 1).astype(jnp.float32)
    ended = (jnp.broadcast_to(ends8, (N_E, MAXB)) <= iom).astype(jnp.float32)
    be_raw = jnp.sum(ended, axis=0, keepdims=True)
    nb = lax.slice(ends8, (N_E - 1, 0), (N_E, 1))
    active = lax.broadcasted_iota(
        jnp.int32, (1, MAXB), 1).astype(jnp.float32) < nb
    be_last = jnp.max(jnp.where(active, be_raw, 0.0))
    be = jnp.minimum(be_raw, be_last)
    be_ref[...] = be.astype(jnp.int32)
    nb_ref[...] = nb.astype(jnp.int32)


def _gating_call(xf, gaw, gab, gbw, gbb):
    return pl.pallas_call(
        _gating_body,
        out_shape=[
            jax.ShapeDtypeStruct((SEQ // 128, 128), jnp.int32),
            jax.ShapeDtypeStruct((1, MAXB), jnp.int32),
            jax.ShapeDtypeStruct((1, 1), jnp.int32),
            jax.ShapeDtypeStruct((1, 1), jnp.float32),
        ],
    )(xf, gaw, gab, gbw, gbb)


_NC, _NS = 2, 16
_NW = _NC * _NS
_BPW = SEQ // _NW


@functools.cache
def _sc_kernels():
    mesh = plsc.VectorSubcoreMesh(core_axis_name="c", subcore_axis_name="s")

    @functools.partial(
        pl.kernel,
        mesh=mesh,
        out_type=jax.ShapeDtypeStruct((NSLOT, D_MODEL), jnp.float32),
        scratch_types=[
            pltpu.VMEM((_BPW,), jnp.int32),
            pltpu.VMEM((_BPW, D_MODEL), jnp.float32),
            pltpu.SemaphoreType.DMA,
        ],
    )
    def dispatch_scatter(x_hbm, slot_hbm, xs_hbm, idx_v, rows_v, sem):
        wid = lax.axis_index("s") * _NC + lax.axis_index("c")
        base = wid * _BPW
        pltpu.sync_copy(slot_hbm.at[pl.ds(base, _BPW)], idx_v)
        pltpu.sync_copy(x_hbm.at[pl.ds(base, _BPW)], rows_v)
        pltpu.async_copy(rows_v, xs_hbm.at[idx_v], sem).wait()

    @functools.partial(
        pl.kernel,
        mesh=mesh,
        out_type=jax.ShapeDtypeStruct((SEQ, D_MODEL), jnp.float32),
        scratch_types=[
            pltpu.VMEM((_BPW,), jnp.int32),
            pltpu.VMEM((_BPW, D_MODEL), jnp.float32),
            pltpu.SemaphoreType.DMA,
        ],
    )
    def combine_gather(ys_hbm, slot_hbm, out_hbm, idx_v, rows_v, sem):
        wid = lax.axis_index("s") * _NC + lax.axis_index("c")
        base = wid * _BPW
        pltpu.sync_copy(slot_hbm.at[pl.ds(base, _BPW)], idx_v)
        pltpu.async_copy(ys_hbm.at[idx_v], rows_v, sem).wait()
        pltpu.sync_copy(rows_v, out_hbm.at[pl.ds(base, _BPW)])

    return dispatch_scatter, combine_gather


def _ffn_body(be_ref, nb_ref, x_ref, w1_ref, b1_ref, w2_ref, b2_ref, o_ref):
    i = pl.program_id(0)

    @pl.when(i < nb_ref[0])
    def _():
        xb = x_ref[...]
        h = jnp.dot(xb, w1_ref[0], preferred_element_type=jnp.float32)
        h = h + b1_ref[0]
        a = h[:, :D_FF]
        g = h[:, D_FF:]
        act = a * (g * lax.logistic(g))
        y = jnp.dot(act, w2_ref[0], preferred_element_type=jnp.float32)
        o_ref[...] = y + b2_ref[0]


def _ffn_call(be, nb, xs, w1, b1, w2, b2):
    grid_spec = pltpu.PrefetchScalarGridSpec(
        num_scalar_prefetch=2,
        grid=(MAXB,),
        in_specs=[
            pl.BlockSpec((BLK, D_MODEL),
                         lambda i, be, nb: (jnp.minimum(i, nb[0] - 1), 0)),
            pl.BlockSpec((1, D_MODEL, 2 * D_FF),
                         lambda i, be, nb: (be[i], 0, 0)),
            pl.BlockSpec((1, 1, 2 * D_FF), lambda i, be, nb: (be[i], 0, 0)),
            pl.BlockSpec((1, D_FF, D_MODEL),
                         lambda i, be, nb: (be[i], 0, 0)),
            pl.BlockSpec((1, 1, D_MODEL), lambda i, be, nb: (be[i], 0, 0)),
        ],
        out_specs=pl.BlockSpec((BLK, D_MODEL),
                               lambda i, be, nb: (jnp.minimum(i, nb[0] - 1), 0)),
    )
    return pl.pallas_call(
        _ffn_body,
        grid_spec=grid_spec,
        out_shape=jax.ShapeDtypeStruct((NSLOT, D_MODEL), jnp.float32),
    )(be, nb, xs, w1, b1, w2, b2)


def kernel(x, gA_w, gA_b, gB_w, gB_b, w1_w, w1_b, w2_w, w2_b):
    b, s, d = x.shape
    xf = x.reshape(s, d)

    slot2d, be2d, nb2d, aux2d = _gating_call(
        xf, gA_w.T, gA_b.reshape(1, N_A), gB_w.T, gB_b.reshape(1, N_B))
    slot = slot2d.reshape(s)
    be = be2d.reshape(MAXB)
    nb = nb2d.reshape(1)

    dispatch_scatter, combine_gather = _sc_kernels()
    xs = dispatch_scatter(xf, slot)

    w1 = w1_w.reshape(N_E, d, 2 * D_FF)
    b1 = w1_b.reshape(N_E, 1, 2 * D_FF)
    w2 = w2_w.reshape(N_E, D_FF, d)
    b2 = w2_b.reshape(N_E, 1, d)
    ys = _ffn_call(be, nb, xs, w1, b1, w2, b2)

    outf = combine_gather(ys, slot)
    return outf.reshape(b, s, d), aux2d.reshape(())
